# baseline (speedup 1.0000x reference)
import functools

import jax
import jax.numpy as jnp
import numpy as np
from jax import lax
from jax.experimental import pallas as pl
from jax.experimental.pallas import tpu as pltpu
from jax.experimental.pallas import tpu_sc as plsc

_INPUT_DIM = 4096
_NUM_EXPERTS = 64
_TOP_K = 8
_NOISE_EPS = 0.01
_BATCH = 8192
_BLK = 512

_NC = 2
_NS = 16
_NW = _NC * _NS
_RPW = _BATCH // _NW
_GRP = 16
_NGRP = _RPW // _GRP
_ILP = 2

_STD_NORMAL = np.asarray(
    jax.random.normal(jax.random.key(42), (_BATCH, _NUM_EXPERTS),
                      dtype=jnp.float32))


def _gating_kernel(x_ref, w_ref, std_ref, bias_ref, gk_ref, idx3_ref,
                   load_ref):
    logits = jnp.dot(x_ref[...], w_ref[...],
                     preferred_element_type=jnp.float32)
    clean = logits[:, :_NUM_EXPERTS]
    raw_noise = logits[:, _NUM_EXPERTS:]
    noise = std_ref[...] * jax.nn.softplus(raw_noise) * _NOISE_EPS
    gates = jax.nn.sigmoid(clean + noise)
    bias_gates = gates + bias_ref[...]

    iota = lax.broadcasted_iota(jnp.int32, (_BLK, _NUM_EXPERTS), 1)
    work = bias_gates
    gk_cols = []
    idx_cols = []
    part = jnp.zeros((1, _NUM_EXPERTS), jnp.float32)
    for _ in range(_TOP_K):
        m = jnp.max(work, axis=1, keepdims=True)
        cand = jnp.where(work == m, iota, _NUM_EXPERTS)
        idx = jnp.min(cand, axis=1, keepdims=True)
        sel = iota == idx
        gk_cols.append(jnp.sum(jnp.where(sel, gates, 0.0), axis=1,
                               keepdims=True))
        idx_cols.append(idx)
        part = part + jnp.sum(sel.astype(jnp.float32), axis=0,
                              keepdims=True)
        work = jnp.where(sel, -jnp.inf, work)

    gk_ref[...] = jnp.concatenate(gk_cols, axis=1)
    idx3_ref[...] = jnp.concatenate(idx_cols, axis=1).astype(jnp.float32)

    @pl.when(pl.program_id(0) == 0)
    def _init():
        load_ref[...] = jnp.zeros_like(load_ref)

    load_ref[...] += part * (1.0 / (_BATCH * _TOP_K))


def _gates_tc(x, w_comb, std, bias_row):
    return pl.pallas_call(
        _gating_kernel,
        grid=(_BATCH // _BLK,),
        in_specs=[
            pl.BlockSpec((_BLK, _INPUT_DIM), lambda i: (i, 0)),
            pl.BlockSpec((_INPUT_DIM, 2 * _NUM_EXPERTS), lambda i: (0, 0)),
            pl.BlockSpec((_BLK, _NUM_EXPERTS), lambda i: (i, 0)),
            pl.BlockSpec((1, _NUM_EXPERTS), lambda i: (0, 0)),
        ],
        out_specs=[
            pl.BlockSpec((_BLK, _TOP_K), lambda i: (i, 0)),
            pl.BlockSpec((_BLK, _TOP_K), lambda i: (i, 0)),
            pl.BlockSpec((1, _NUM_EXPERTS), lambda i: (0, 0)),
        ],
        out_shape=[
            jax.ShapeDtypeStruct((_BATCH, _TOP_K), jnp.float32),
            jax.ShapeDtypeStruct((_BATCH, _TOP_K), jnp.float32),
            jax.ShapeDtypeStruct((1, _NUM_EXPERTS), jnp.float32),
        ],
        compiler_params=pltpu.CompilerParams(
            dimension_semantics=("arbitrary",)),
    )(x, w_comb, std, bias_row)


_SC_MESH = plsc.VectorSubcoreMesh(core_axis_name="c", subcore_axis_name="s")

_GATHER_DN = lax.GatherDimensionNumbers(
    offset_dims=(), collapsed_slice_dims=(0,), start_index_map=(0,))


def _vgather(vec, idx):
    return lax.gather(vec, idx.reshape(16, 1), _GATHER_DN, (1,),
                      mode=lax.GatherScatterMode.PROMISE_IN_BOUNDS)


@functools.partial(
    pl.kernel,
    mesh=_SC_MESH,
    out_type=[
        jax.ShapeDtypeStruct((_BATCH // (_ILP * _GRP), _ILP * _GRP,
                              _NUM_EXPERTS), jnp.int32),
    ],
    scratch_types=[
        pltpu.VMEM((_RPW // 2, 2 * _TOP_K), jnp.float32),
        pltpu.VMEM((_ILP * _GRP, _NUM_EXPERTS), jnp.int32),
    ],
)
def _scatter_sc(idx3_hbm, r3_hbm, slab, rg):
    wid = lax.axis_index("s") * _NC + lax.axis_index("c")
    pltpu.sync_copy(idx3_hbm.at[wid], slab)

    lane = lax.iota(jnp.int32, 16)
    chunk_iotas = [lane + c * 16 for c in range(4)]
    ones_i = jnp.ones((16,), jnp.int32)
    zeros_i = jnp.zeros((16,), jnp.int32)

    def group_body(g, carry):
        for h in range(_ILP):
            gg = g * _ILP + h
            for rr in range(_GRP // 2):
                vi = slab[gg * (_GRP // 2) + rr, pl.ds(0, 16)].astype(jnp.int32)
                for half in range(2):
                    sp = [_vgather(vi, jnp.full((16,), half * _TOP_K + j,
                                                jnp.int32))
                          for j in range(_TOP_K)]
                    for c in range(4):
                        hit = jnp.where(sp[0] == chunk_iotas[c], ones_i,
                                        zeros_i)
                        for j in range(1, _TOP_K):
                            hit = hit | jnp.where(sp[j] == chunk_iotas[c],
                                                  ones_i, zeros_i)
                        rg[h * _GRP + 2 * rr + half,
                           pl.ds(c * 16, 16)] = hit

        gid = wid * (_NGRP // _ILP) + g
        pltpu.sync_copy(rg, r3_hbm.at[gid])
        return carry

    lax.fori_loop(0, _NGRP // _ILP, group_body, 0)


def kernel(x, w_gate, w_noise, bias):
    w_comb = jnp.concatenate([w_gate, w_noise], axis=0).T
    std = jnp.asarray(_STD_NORMAL)
    gk, idxm, load = _gates_tc(x, w_comb, std,
                               bias.reshape(1, _NUM_EXPERTS))
    idx3 = idxm.reshape(_NW, _RPW // 2, 2 * _TOP_K)
    (r3,) = _scatter_sc(idx3)
    router = r3.reshape(_BATCH, _NUM_EXPERTS)
    return gk, router, load.reshape(_NUM_EXPERTS)


# speedup vs baseline: 1.2109x; 1.2109x over previous
import functools

import jax
import jax.numpy as jnp
import numpy as np
from jax import lax
from jax.experimental import pallas as pl
from jax.experimental.pallas import tpu as pltpu
from jax.experimental.pallas import tpu_sc as plsc

_INPUT_DIM = 4096
_NUM_EXPERTS = 64
_TOP_K = 8
_NOISE_EPS = 0.01
_BATCH = 8192
_BLK = 512

_NC = 2
_NS = 16
_NW = _NC * _NS
_RPW = _BATCH // _NW
_GRP = 16
_NGRP = _RPW // _GRP
_ILP = 2

_STD_NORMAL = np.asarray(
    jax.random.normal(jax.random.key(42), (_BATCH, _NUM_EXPERTS),
                      dtype=jnp.float32))


def _gates_kernel(x_ref, w_ref, std_ref, bias_ref, bg3_ref):
    logits = jnp.dot(x_ref[...], w_ref[...],
                     preferred_element_type=jnp.float32)
    clean = logits[:, :_NUM_EXPERTS]
    raw_noise = logits[:, _NUM_EXPERTS:]
    noise = std_ref[...] * jax.nn.softplus(raw_noise) * _NOISE_EPS
    gates = jax.nn.sigmoid(clean + noise)
    bias_gates = gates + bias_ref[...]
    for h in range(_BLK // _RPW):
        bg3_ref[h] = bias_gates[h * _RPW:(h + 1) * _RPW, :].T


def _gates_tc(x, w_comb, std, bias_row):
    return pl.pallas_call(
        _gates_kernel,
        grid=(_BATCH // _BLK,),
        in_specs=[
            pl.BlockSpec((_BLK, _INPUT_DIM), lambda i: (i, 0)),
            pl.BlockSpec((_INPUT_DIM, 2 * _NUM_EXPERTS), lambda i: (0, 0)),
            pl.BlockSpec((_BLK, _NUM_EXPERTS), lambda i: (i, 0)),
            pl.BlockSpec((1, _NUM_EXPERTS), lambda i: (0, 0)),
        ],
        out_specs=pl.BlockSpec((_BLK // _RPW, _NUM_EXPERTS, _RPW),
                               lambda i: (i, 0, 0)),
        out_shape=jax.ShapeDtypeStruct((_NW, _NUM_EXPERTS, _RPW),
                                       jnp.float32),
        compiler_params=pltpu.CompilerParams(
            dimension_semantics=("arbitrary",)),
    )(x, w_comb, std, bias_row)


_SC_MESH = plsc.VectorSubcoreMesh(core_axis_name="c", subcore_axis_name="s")

_GATHER_DN = lax.GatherDimensionNumbers(
    offset_dims=(), collapsed_slice_dims=(0,), start_index_map=(0,))


def _vgather(vec, idx):
    return lax.gather(vec, idx.reshape(16, 1), _GATHER_DN, (1,),
                      mode=lax.GatherScatterMode.PROMISE_IN_BOUNDS)


@functools.partial(
    pl.kernel,
    mesh=_SC_MESH,
    out_type=[
        jax.ShapeDtypeStruct((_BATCH // (_ILP * _GRP), _ILP * _GRP, _GRP),
                             jnp.float32),
        jax.ShapeDtypeStruct((_BATCH // (_ILP * _GRP), _ILP * _GRP,
                              _NUM_EXPERTS), jnp.int32),
        jax.ShapeDtypeStruct((_NW, 4, _GRP), jnp.int32),
    ],
    scratch_types=[
        pltpu.VMEM((_NUM_EXPERTS, _RPW), jnp.float32),
        pltpu.VMEM((_NUM_EXPERTS,), jnp.float32),
        pltpu.VMEM((_ILP * _GRP, _NUM_EXPERTS), jnp.int32),
        pltpu.VMEM((_ILP * _GRP, _GRP), jnp.float32),
        pltpu.VMEM((4, _GRP), jnp.int32),
    ],
)
def _route_sc(bg3_hbm, bias_hbm, gk3_hbm, r3_hbm, counts_hbm,
              slab, bias_v, rg, gkg, cnt):
    wid = lax.axis_index("s") * _NC + lax.axis_index("c")
    pltpu.sync_copy(bg3_hbm.at[wid], slab)
    pltpu.sync_copy(bias_hbm, bias_v)
    bias_regs = [bias_v[pl.ds(c * 16, 16)] for c in range(_NUM_EXPERTS // 16)]

    lane = lax.iota(jnp.int32, 16)
    zeros_i = jnp.zeros((16,), jnp.int32)
    neg_inf = jnp.full((16,), -jnp.inf, jnp.float32)
    for c in range(4):
        cnt[c, :] = zeros_i

    def group_body(g, carry):
        col0 = g * (_GRP * _ILP)

        def insert(e, kcarry):
            out = []
            eix = jnp.full((16,), e, jnp.int32)
            for h in range(_ILP):
                ks = list(kcarry[2 * _TOP_K * h:2 * _TOP_K * h + _TOP_K])
                ix = list(kcarry[2 * _TOP_K * h + _TOP_K:
                                 2 * _TOP_K * (h + 1)])
                x = slab[e, pl.ds(col0 + h * _GRP, _GRP)]
                gt = [x > ks[j] for j in range(_TOP_K)]
                nks = [jnp.where(gt[0], x, ks[0])]
                nix = [jnp.where(gt[0], eix, ix[0])]
                for j in range(1, _TOP_K):
                    sk = jnp.where(gt[j - 1], ks[j - 1], x)
                    si = jnp.where(gt[j - 1], ix[j - 1], eix)
                    nks.append(jnp.where(gt[j], sk, ks[j]))
                    nix.append(jnp.where(gt[j], si, ix[j]))
                out += nks + nix
            return tuple(out)

        kcarry0 = ((neg_inf,) * _TOP_K + (zeros_i,) * _TOP_K) * _ILP
        res = lax.fori_loop(0, _NUM_EXPERTS, insert, kcarry0)

        for h in range(_ILP):
            ks = res[2 * _TOP_K * h:2 * _TOP_K * h + _TOP_K]
            ix = res[2 * _TOP_K * h + _TOP_K:2 * _TOP_K * (h + 1)]

            lo = zeros_i
            hi = zeros_i
            for j in range(_TOP_K):
                bit = jnp.int32(1) << (ix[j] & 15)
                bit16 = jnp.where((ix[j] & 16) != 0, bit << 16, bit)
                lo = lo | jnp.where(ix[j] < 32, bit16, 0)
                hi = hi | jnp.where(ix[j] >= 32, bit16, 0)

            gkv = []
            for j in range(_TOP_K):
                sel = ix[j] >> 4
                b = _vgather(bias_regs[0], ix[j] & 15)
                for c in range(1, _NUM_EXPERTS // 16):
                    b = jnp.where(sel == c,
                                  _vgather(bias_regs[c], ix[j] & 15), b)
                gkv.append(ks[j] - b)

            cacc = [zeros_i] * 4
            for r in range(_GRP):
                rsplat = jnp.full((16,), r, jnp.int32)
                lo_r = _vgather(lo, rsplat)
                hi_r = _vgather(hi, rsplat)
                for c in range(4):
                    src = lo_r if c < 2 else hi_r
                    onehot = (src >> (lane + (c % 2) * 16)) & 1
                    rg[h * _GRP + r, pl.ds(c * 16, 16)] = onehot
                    cacc[c] = cacc[c] + onehot
                gvals = jnp.zeros((16,), jnp.float32)
                for j in range(_TOP_K):
                    gvals = jnp.where(lane == j, _vgather(gkv[j], rsplat),
                                      gvals)
                gkg[h * _GRP + r, :] = gvals
            for c in range(4):
                cnt[c, :] = cnt[c, :] + cacc[c]

        gid = wid * (_NGRP // _ILP) + g
        pltpu.sync_copy(rg, r3_hbm.at[gid])
        pltpu.sync_copy(gkg, gk3_hbm.at[gid])
        return carry

    lax.fori_loop(0, _NGRP // _ILP, group_body, 0)
    pltpu.sync_copy(cnt, counts_hbm.at[wid])


def kernel(x, w_gate, w_noise, bias):
    w_comb = jnp.concatenate([w_gate, w_noise], axis=0).T
    std = jnp.asarray(_STD_NORMAL)
    bg3 = _gates_tc(x, w_comb, std, bias.reshape(1, _NUM_EXPERTS))
    gk3, r3, counts = _route_sc(bg3, bias)
    gk = gk3.reshape(_BATCH, _GRP)[:, :_TOP_K]
    router = r3.reshape(_BATCH, _NUM_EXPERTS)
    load = (jnp.sum(counts, axis=0).astype(jnp.float32).reshape(_NUM_EXPERTS)
            * (1.0 / (_BATCH * _TOP_K)))
    return gk, router, load
